# Initial kernel scaffold; baseline (speedup 1.0000x reference)
import jax
import jax.numpy as jnp
from jax.experimental import pallas as pl


def kernel(x, W_ctl, b_ctl, emb, W_comp, b_comp):
    raise NotImplementedError("write your pallas kernel here")


# speedup vs baseline: 1.4955x; 1.4955x over previous
import jax
import jax.numpy as jnp
from jax import lax
from jax.experimental import pallas as pl
from jax.experimental.pallas import tpu as pltpu

DEPTH = 2
DIM_EMB = 128
N_MODULES = 8


def _argmin8(score):
    k = score.shape[-1]
    min_s = jnp.min(score)
    iota = lax.broadcasted_iota(jnp.int32, score.shape, 1)
    return jnp.min(jnp.where(score == min_s, iota, k))


def _fused_kernel(x_ref, wctl_ref, bctl_ref, emb_ref, embc_ref,
                  wcomp_ref, bcomp_ref, y_ref, ctl_ref, ctln_ref):
    x_e = x_ref[0]
    hw = x_e.shape[1]
    xm = jnp.sum(x_e, axis=1, keepdims=True) * (1.0 / hw)
    ctl_col = jnp.dot(wctl_ref[...], xm,
                      preferred_element_type=jnp.float32) + bctl_ref[...]
    e2 = jnp.sum(emb_ref[...] ** 2, axis=0, keepdims=True)

    def route(t):
        ctl_t = ctl_col[t * DIM_EMB:(t + 1) * DIM_EMB, :]
        dots = lax.dot_general(ctl_t, emb_ref[...], (((0,), (0,)), ((), ())),
                               preferred_element_type=jnp.float32)
        score = e2 - 2.0 * dots
        return ctl_t, _argmin8(score)

    ctl_0, idx0 = route(0)
    ctl_1, idx1 = route(1)

    ctl_ref[0, :, 0:1] = ctl_0
    ctl_ref[0, :, 1:2] = ctl_1
    ctln_ref[0, :, 0:1] = embc_ref[idx0]
    ctln_ref[0, :, 1:2] = embc_ref[idx1]

    w1 = wcomp_ref[idx0]
    w2 = wcomp_ref[idx1]
    b1 = bcomp_ref[idx0]
    b2 = bcomp_ref[idx1]
    w_eff = jnp.dot(w2, w1, preferred_element_type=jnp.float32)
    b_eff = jnp.dot(w2, b1, preferred_element_type=jnp.float32) + b2
    y_ref[0] = jnp.dot(w_eff, x_e,
                       preferred_element_type=jnp.float32) + b_eff


def kernel(x, W_ctl, b_ctl, emb, W_comp, b_comp):
    Bn, C, H, W = x.shape
    HW = H * W
    x2 = x.reshape(Bn, C, HW)
    W_ctl_dm = (W_ctl.reshape(DIM_EMB, DEPTH, C)
                .transpose(1, 0, 2).reshape(DEPTH * DIM_EMB, C))
    b_ctl_dm = b_ctl.reshape(DIM_EMB, DEPTH).T.reshape(DEPTH * DIM_EMB, 1)
    emb_cols = emb.T.reshape(N_MODULES, DIM_EMB, 1)
    b_comp_c = b_comp.reshape(N_MODULES, C, 1)

    grid = (Bn,)
    y, ctl, ctln = pl.pallas_call(
        _fused_kernel,
        grid=grid,
        in_specs=[
            pl.BlockSpec((1, C, HW), lambda e: (e, 0, 0)),
            pl.BlockSpec((DEPTH * DIM_EMB, C), lambda e: (0, 0)),
            pl.BlockSpec((DEPTH * DIM_EMB, 1), lambda e: (0, 0)),
            pl.BlockSpec((DIM_EMB, N_MODULES), lambda e: (0, 0)),
            pl.BlockSpec((N_MODULES, DIM_EMB, 1), lambda e: (0, 0, 0)),
            pl.BlockSpec((N_MODULES, C, C), lambda e: (0, 0, 0)),
            pl.BlockSpec((N_MODULES, C, 1), lambda e: (0, 0, 0)),
        ],
        out_specs=[
            pl.BlockSpec((1, C, HW), lambda e: (e, 0, 0)),
            pl.BlockSpec((1, DIM_EMB, DEPTH), lambda e: (e, 0, 0)),
            pl.BlockSpec((1, DIM_EMB, DEPTH), lambda e: (e, 0, 0)),
        ],
        out_shape=[
            jax.ShapeDtypeStruct((Bn, C, HW), jnp.float32),
            jax.ShapeDtypeStruct((Bn, DIM_EMB, DEPTH), jnp.float32),
            jax.ShapeDtypeStruct((Bn, DIM_EMB, DEPTH), jnp.float32),
        ],
        compiler_params=pltpu.CompilerParams(
            dimension_semantics=("arbitrary",),
        ),
    )(x2, W_ctl_dm, b_ctl_dm, emb, emb_cols, W_comp, b_comp_c)
    return (y.reshape(Bn, C, H, W), ctl, ctln)
